# baseline (speedup 1.0000x reference)
import functools

import jax
import jax.numpy as jnp
from jax import lax
from jax.experimental import pallas as pl
from jax.experimental.pallas import tpu as pltpu
from jax.experimental.pallas import tpu_sc as plsc

T = 8192
B = 2048
HIDDEN = 2048
FFN = 7168

_NC = 2
_NS = 16
_NW = _NC * _NS
_BPW = B // _NW
_CHUNK = 16
_NCHUNK = _BPW // _CHUNK


def _sc_gather(table, idx):
    mesh = plsc.VectorSubcoreMesh(core_axis_name="c", subcore_axis_name="s")

    @functools.partial(
        pl.kernel,
        out_type=jax.ShapeDtypeStruct((B, HIDDEN), jnp.float32),
        mesh=mesh,
        scratch_types=[
            pltpu.VMEM((_BPW,), jnp.int32),
            pltpu.VMEM((_CHUNK, HIDDEN), jnp.float32),
            pltpu.VMEM((_CHUNK, HIDDEN), jnp.float32),
            pltpu.SemaphoreType.DMA,
            pltpu.SemaphoreType.DMA,
        ],
    )
    def gather_kernel(table_hbm, idx_hbm, out_hbm, idx_v, rows0, rows1, s0, s1):
        wid = lax.axis_index("s") * _NC + lax.axis_index("c")
        base = wid * _BPW
        pltpu.sync_copy(idx_hbm.at[pl.ds(base, _BPW)], idx_v)
        bufs = (rows0, rows1)
        sems = (s0, s1)
        pltpu.async_copy(table_hbm.at[idx_v.at[pl.ds(0, _CHUNK)]], bufs[0], sems[0])
        for c in range(_NCHUNK):
            if c + 1 < _NCHUNK:
                pltpu.async_copy(
                    table_hbm.at[idx_v.at[pl.ds((c + 1) * _CHUNK, _CHUNK)]],
                    bufs[(c + 1) % 2],
                    sems[(c + 1) % 2],
                )
            pltpu.make_async_copy(
                table_hbm.at[idx_v.at[pl.ds(c * _CHUNK, _CHUNK)]],
                bufs[c % 2],
                sems[c % 2],
            ).wait()
            pltpu.sync_copy(bufs[c % 2], out_hbm.at[pl.ds(base + c * _CHUNK, _CHUNK)])

    return gather_kernel(table, idx)


_NT = (((1,), (1,)), ((), ()))

_FB = 512
_NFB = FFN // _FB
_MB = 512
_NMB = B // _MB


def _w2t_body(w2_ref, w2o_ref):
    w2o_ref[...] = w2_ref[...].astype(jnp.bfloat16).T


def _w2t(W2):
    return pl.pallas_call(
        _w2t_body,
        grid=(_NFB,),
        in_specs=[pl.BlockSpec((HIDDEN, _FB), lambda k: (0, k))],
        out_specs=pl.BlockSpec((_FB, HIDDEN), lambda k: (k, 0)),
        out_shape=jax.ShapeDtypeStruct((FFN, HIDDEN), jnp.bfloat16),
        compiler_params=pltpu.CompilerParams(
            dimension_semantics=("parallel",),
        ),
    )(W2)


def _gate_body(g_ref, w1_ref, w3_ref, rw_ref, cur_ref):
    g = g_ref[...]
    w1 = w1_ref[...].astype(jnp.bfloat16)
    w3 = w3_ref[...].astype(jnp.bfloat16)
    h1 = lax.dot_general(g, w1, _NT, preferred_element_type=jnp.float32)
    h3 = lax.dot_general(g, w3, _NT, preferred_element_type=jnp.float32)
    cur = h1 * jax.nn.sigmoid(h1) * h3 * rw_ref[...]
    cur_ref[...] = cur.astype(jnp.bfloat16)


def _down_body(cur_ref, w2_ref, out_ref):
    cur = cur_ref[...]
    w2t = w2_ref[...]
    out_ref[...] = lax.dot_general(
        cur, w2t, (((1,), (0,)), ((), ())), preferred_element_type=jnp.float32
    )


def _ffn(g16, rw_col, W1, W2, W3, w2_16):
    cur16 = pl.pallas_call(
        _gate_body,
        grid=(_NFB,),
        in_specs=[
            pl.BlockSpec((B, HIDDEN), lambda k: (0, 0)),
            pl.BlockSpec((_FB, HIDDEN), lambda k: (k, 0)),
            pl.BlockSpec((_FB, HIDDEN), lambda k: (k, 0)),
            pl.BlockSpec((B, 1), lambda k: (0, 0)),
        ],
        out_specs=pl.BlockSpec((B, _FB), lambda k: (0, k)),
        out_shape=jax.ShapeDtypeStruct((B, FFN), jnp.bfloat16),
        compiler_params=pltpu.CompilerParams(
            dimension_semantics=("parallel",),
        ),
    )(g16, W1, W3, rw_col)
    return pl.pallas_call(
        _down_body,
        grid=(_NMB,),
        in_specs=[
            pl.BlockSpec((_MB, FFN), lambda m: (m, 0)),
            pl.BlockSpec((FFN, HIDDEN), lambda m: (0, 0)),
        ],
        out_specs=pl.BlockSpec((_MB, HIDDEN), lambda m: (m, 0)),
        out_shape=jax.ShapeDtypeStruct((B, HIDDEN), jnp.float32),
        compiler_params=pltpu.CompilerParams(
            dimension_semantics=("parallel",),
            vmem_limit_bytes=128 * 1024 * 1024,
        ),
    )(cur16, w2_16)


def kernel(hidden_states, input_idx, routing_weights, W1, W2, W3):
    idx = input_idx.astype(jnp.int32)
    w2_16 = _w2t(W2)
    gathered = _sc_gather(hidden_states, idx)
    g16 = gathered.astype(jnp.bfloat16)
    rw_col = routing_weights.reshape(B, 1)
    return _ffn(g16, rw_col, W1, W2, W3, w2_16)


# speedup vs baseline: 1.0787x; 1.0787x over previous
import functools

import jax
import jax.numpy as jnp
from jax import lax
from jax.experimental import pallas as pl
from jax.experimental.pallas import tpu as pltpu
from jax.experimental.pallas import tpu_sc as plsc

T = 8192
B = 2048
HIDDEN = 2048
FFN = 7168

_NC = 2
_NS = 16
_NW = _NC * _NS
_BPW = B // _NW
_CHUNK = 16
_NCHUNK = _BPW // _CHUNK


def _sc_gather(table, idx):
    mesh = plsc.VectorSubcoreMesh(core_axis_name="c", subcore_axis_name="s")

    @functools.partial(
        pl.kernel,
        out_type=jax.ShapeDtypeStruct((B, HIDDEN), jnp.float32),
        mesh=mesh,
        scratch_types=[
            pltpu.VMEM((_BPW,), jnp.int32),
            pltpu.VMEM((_CHUNK, HIDDEN), jnp.float32),
            pltpu.VMEM((_CHUNK, HIDDEN), jnp.float32),
            pltpu.SemaphoreType.DMA,
            pltpu.SemaphoreType.DMA,
        ],
    )
    def gather_kernel(table_hbm, idx_hbm, out_hbm, idx_v, rows0, rows1, s0, s1):
        wid = lax.axis_index("s") * _NC + lax.axis_index("c")
        base = wid * _BPW
        pltpu.sync_copy(idx_hbm.at[pl.ds(base, _BPW)], idx_v)
        bufs = (rows0, rows1)
        sems = (s0, s1)
        pltpu.async_copy(table_hbm.at[idx_v.at[pl.ds(0, _CHUNK)]], bufs[0], sems[0])
        for c in range(_NCHUNK):
            if c + 1 < _NCHUNK:
                pltpu.async_copy(
                    table_hbm.at[idx_v.at[pl.ds((c + 1) * _CHUNK, _CHUNK)]],
                    bufs[(c + 1) % 2],
                    sems[(c + 1) % 2],
                )
            pltpu.make_async_copy(
                table_hbm.at[idx_v.at[pl.ds(c * _CHUNK, _CHUNK)]],
                bufs[c % 2],
                sems[c % 2],
            ).wait()
            pltpu.sync_copy(bufs[c % 2], out_hbm.at[pl.ds(base + c * _CHUNK, _CHUNK)])

    return gather_kernel(table, idx)


_NT = (((1,), (1,)), ((), ()))

_FB = 512
_NFB = FFN // _FB
_MB = 512
_NMB = B // _MB


def _gate_body(g_ref, w1_ref, w3_ref, rw_ref, w2_ref, cur_ref, w2o_ref):
    g = g_ref[...]
    w1 = w1_ref[...].astype(jnp.bfloat16)
    w3 = w3_ref[...].astype(jnp.bfloat16)
    h1 = lax.dot_general(g, w1, _NT, preferred_element_type=jnp.float32)
    h3 = lax.dot_general(g, w3, _NT, preferred_element_type=jnp.float32)
    cur = h1 * jax.nn.sigmoid(h1) * h3 * rw_ref[...]
    cur_ref[...] = cur.astype(jnp.bfloat16)
    w2o_ref[...] = w2_ref[...].astype(jnp.bfloat16).T


def _down_body(cur_ref, w2_ref, out_ref):
    cur = cur_ref[...]
    w2t = w2_ref[...]
    out_ref[...] = lax.dot_general(
        cur, w2t, (((1,), (0,)), ((), ())), preferred_element_type=jnp.float32
    )


def _ffn(g16, rw_col, W1, W2, W3):
    cur16, w2_16 = pl.pallas_call(
        _gate_body,
        grid=(_NFB,),
        in_specs=[
            pl.BlockSpec((B, HIDDEN), lambda k: (0, 0)),
            pl.BlockSpec((_FB, HIDDEN), lambda k: (k, 0)),
            pl.BlockSpec((_FB, HIDDEN), lambda k: (k, 0)),
            pl.BlockSpec((B, 1), lambda k: (0, 0)),
            pl.BlockSpec((HIDDEN, _FB), lambda k: (0, k)),
        ],
        out_specs=[
            pl.BlockSpec((B, _FB), lambda k: (0, k)),
            pl.BlockSpec((_FB, HIDDEN), lambda k: (k, 0)),
        ],
        out_shape=[
            jax.ShapeDtypeStruct((B, FFN), jnp.bfloat16),
            jax.ShapeDtypeStruct((FFN, HIDDEN), jnp.bfloat16),
        ],
        compiler_params=pltpu.CompilerParams(
            dimension_semantics=("parallel",),
        ),
    )(g16, W1, W3, rw_col, W2)
    return pl.pallas_call(
        _down_body,
        grid=(_NMB,),
        in_specs=[
            pl.BlockSpec((_MB, FFN), lambda m: (m, 0)),
            pl.BlockSpec((FFN, HIDDEN), lambda m: (0, 0)),
        ],
        out_specs=pl.BlockSpec((_MB, HIDDEN), lambda m: (m, 0)),
        out_shape=jax.ShapeDtypeStruct((B, HIDDEN), jnp.float32),
        compiler_params=pltpu.CompilerParams(
            dimension_semantics=("parallel",),
            vmem_limit_bytes=128 * 1024 * 1024,
        ),
    )(cur16, w2_16)


def kernel(hidden_states, input_idx, routing_weights, W1, W2, W3):
    idx = input_idx.astype(jnp.int32)
    gathered = _sc_gather(hidden_states, idx)
    g16 = gathered.astype(jnp.bfloat16)
    rw_col = routing_weights.reshape(B, 1)
    return _ffn(g16, rw_col, W1, W2, W3)


# speedup vs baseline: 1.0862x; 1.0070x over previous
import functools

import jax
import jax.numpy as jnp
from jax import lax
from jax.experimental import pallas as pl
from jax.experimental.pallas import tpu as pltpu
from jax.experimental.pallas import tpu_sc as plsc

T = 8192
B = 2048
HIDDEN = 2048
FFN = 7168

_NC = 2
_NS = 16
_NW = _NC * _NS
_BPW = B // _NW
_CHUNK = 16
_NCHUNK = _BPW // _CHUNK


def _sc_gather(table, idx):
    mesh = plsc.VectorSubcoreMesh(core_axis_name="c", subcore_axis_name="s")

    @functools.partial(
        pl.kernel,
        out_type=jax.ShapeDtypeStruct((B, HIDDEN), jnp.float32),
        mesh=mesh,
        scratch_types=[
            pltpu.VMEM((_BPW,), jnp.int32),
            pltpu.VMEM((_CHUNK, HIDDEN), jnp.float32),
            pltpu.VMEM((_CHUNK, HIDDEN), jnp.float32),
            pltpu.SemaphoreType.DMA,
            pltpu.SemaphoreType.DMA,
        ],
    )
    def gather_kernel(table_hbm, idx_hbm, out_hbm, idx_v, rows0, rows1, s0, s1):
        wid = lax.axis_index("s") * _NC + lax.axis_index("c")
        base = wid * _BPW
        pltpu.sync_copy(idx_hbm.at[pl.ds(base, _BPW)], idx_v)
        bufs = (rows0, rows1)
        sems = (s0, s1)
        pltpu.async_copy(table_hbm.at[idx_v.at[pl.ds(0, _CHUNK)]], bufs[0], sems[0])
        for c in range(_NCHUNK):
            if c + 1 < _NCHUNK:
                pltpu.async_copy(
                    table_hbm.at[idx_v.at[pl.ds((c + 1) * _CHUNK, _CHUNK)]],
                    bufs[(c + 1) % 2],
                    sems[(c + 1) % 2],
                )
            pltpu.make_async_copy(
                table_hbm.at[idx_v.at[pl.ds(c * _CHUNK, _CHUNK)]],
                bufs[c % 2],
                sems[c % 2],
            ).wait()
            pltpu.sync_copy(bufs[c % 2], out_hbm.at[pl.ds(base + c * _CHUNK, _CHUNK)])

    return gather_kernel(table, idx)


_NT = (((1,), (1,)), ((), ()))

_FB = 512
_NFB = FFN // _FB
_MB = 512
_NMB = B // _MB


def _gate_body(g_ref, w1_ref, w3_ref, rw_ref, w2_ref, cur_ref, w2o_ref):
    g = g_ref[...]
    w1 = w1_ref[...].astype(jnp.bfloat16)
    w3 = w3_ref[...].astype(jnp.bfloat16)
    h1 = lax.dot_general(g, w1, _NT, preferred_element_type=jnp.float32)
    h3 = lax.dot_general(g, w3, _NT, preferred_element_type=jnp.float32)
    cur = h1 * jax.nn.sigmoid(h1) * h3 * rw_ref[...]
    cur_ref[...] = cur.astype(jnp.bfloat16)
    w2o_ref[...] = w2_ref[...].astype(jnp.bfloat16)


def _down_body(cur_ref, w2_ref, out_ref):
    cur = cur_ref[...]
    w2 = w2_ref[...]
    out_ref[...] = lax.dot_general(
        cur, w2, _NT, preferred_element_type=jnp.float32
    )


def _ffn(g16, rw_col, W1, W2, W3):
    cur16, w2_16 = pl.pallas_call(
        _gate_body,
        grid=(_NFB,),
        in_specs=[
            pl.BlockSpec((B, HIDDEN), lambda k: (0, 0)),
            pl.BlockSpec((_FB, HIDDEN), lambda k: (k, 0)),
            pl.BlockSpec((_FB, HIDDEN), lambda k: (k, 0)),
            pl.BlockSpec((B, 1), lambda k: (0, 0)),
            pl.BlockSpec((HIDDEN, _FB), lambda k: (0, k)),
        ],
        out_specs=[
            pl.BlockSpec((B, _FB), lambda k: (0, k)),
            pl.BlockSpec((HIDDEN, _FB), lambda k: (0, k)),
        ],
        out_shape=[
            jax.ShapeDtypeStruct((B, FFN), jnp.bfloat16),
            jax.ShapeDtypeStruct((HIDDEN, FFN), jnp.bfloat16),
        ],
        compiler_params=pltpu.CompilerParams(
            dimension_semantics=("parallel",),
            allow_input_fusion=(True, False, False, False, False),
        ),
    )(g16, W1, W3, rw_col, W2)
    return pl.pallas_call(
        _down_body,
        grid=(_NMB,),
        in_specs=[
            pl.BlockSpec((_MB, FFN), lambda m: (m, 0)),
            pl.BlockSpec((HIDDEN, FFN), lambda m: (0, 0)),
        ],
        out_specs=pl.BlockSpec((_MB, HIDDEN), lambda m: (m, 0)),
        out_shape=jax.ShapeDtypeStruct((B, HIDDEN), jnp.float32),
        compiler_params=pltpu.CompilerParams(
            dimension_semantics=("parallel",),
        ),
    )(cur16, w2_16)


def kernel(hidden_states, input_idx, routing_weights, W1, W2, W3):
    idx = input_idx.astype(jnp.int32)
    gathered = _sc_gather(hidden_states, idx)
    g16 = gathered.astype(jnp.bfloat16)
    rw_col = routing_weights.reshape(B, 1)
    return _ffn(g16, rw_col, W1, W2, W3)


# speedup vs baseline: 1.0931x; 1.0064x over previous
import functools

import jax
import jax.numpy as jnp
from jax import lax
from jax.experimental import pallas as pl
from jax.experimental.pallas import tpu as pltpu
from jax.experimental.pallas import tpu_sc as plsc

T = 8192
B = 2048
HIDDEN = 2048
FFN = 7168

_NC = 2
_NS = 16
_NW = _NC * _NS
_BPW = B // _NW
_CHUNK = 16
_NCHUNK = _BPW // _CHUNK


def _sc_gather(table, idx):
    mesh = plsc.VectorSubcoreMesh(core_axis_name="c", subcore_axis_name="s")

    @functools.partial(
        pl.kernel,
        out_type=jax.ShapeDtypeStruct((B, HIDDEN), jnp.float32),
        mesh=mesh,
        scratch_types=[
            pltpu.VMEM((_BPW,), jnp.int32),
            pltpu.VMEM((_CHUNK, HIDDEN), jnp.float32),
            pltpu.VMEM((_CHUNK, HIDDEN), jnp.float32),
            pltpu.SemaphoreType.DMA,
            pltpu.SemaphoreType.DMA,
        ],
    )
    def gather_kernel(table_hbm, idx_hbm, out_hbm, idx_v, rows0, rows1, s0, s1):
        wid = lax.axis_index("s") * _NC + lax.axis_index("c")
        base = wid * _BPW
        pltpu.sync_copy(idx_hbm.at[pl.ds(base, _BPW)], idx_v)
        bufs = (rows0, rows1)
        sems = (s0, s1)
        pltpu.async_copy(table_hbm.at[idx_v.at[pl.ds(0, _CHUNK)]], bufs[0], sems[0])
        for c in range(_NCHUNK):
            if c + 1 < _NCHUNK:
                pltpu.async_copy(
                    table_hbm.at[idx_v.at[pl.ds((c + 1) * _CHUNK, _CHUNK)]],
                    bufs[(c + 1) % 2],
                    sems[(c + 1) % 2],
                )
            pltpu.make_async_copy(
                table_hbm.at[idx_v.at[pl.ds(c * _CHUNK, _CHUNK)]],
                bufs[c % 2],
                sems[c % 2],
            ).wait()
            pltpu.sync_copy(bufs[c % 2], out_hbm.at[pl.ds(base + c * _CHUNK, _CHUNK)])

    return gather_kernel(table, idx)


_NT = (((1,), (1,)), ((), ()))

_FB = 512
_NFB = FFN // _FB
_MB = 512
_NMB = B // _MB


def _gate_body(g_ref, w1_ref, w3_ref, rw_ref, cur_ref):
    g = g_ref[...]
    w1 = w1_ref[...].astype(jnp.bfloat16)
    w3 = w3_ref[...].astype(jnp.bfloat16)
    h1 = lax.dot_general(g, w1, _NT, preferred_element_type=jnp.float32)
    h3 = lax.dot_general(g, w3, _NT, preferred_element_type=jnp.float32)
    cur = h1 * jax.nn.sigmoid(h1) * h3 * rw_ref[...]
    cur_ref[...] = cur.astype(jnp.bfloat16)


_HB = 256
_NHB = HIDDEN // _HB


def _down_body(cur_ref, w2_ref, out_ref):
    w2 = w2_ref[...].astype(jnp.bfloat16)
    _B2 = B // 2
    out_ref[:_B2, :] = lax.dot_general(
        cur_ref[:_B2, :], w2, _NT, preferred_element_type=jnp.float32
    )
    out_ref[_B2:, :] = lax.dot_general(
        cur_ref[_B2:, :], w2, _NT, preferred_element_type=jnp.float32
    )


def _ffn(g16, rw_col, W1, W2, W3):
    cur16 = pl.pallas_call(
        _gate_body,
        grid=(_NFB,),
        in_specs=[
            pl.BlockSpec((B, HIDDEN), lambda k: (0, 0)),
            pl.BlockSpec((_FB, HIDDEN), lambda k: (k, 0)),
            pl.BlockSpec((_FB, HIDDEN), lambda k: (k, 0)),
            pl.BlockSpec((B, 1), lambda k: (0, 0)),
        ],
        out_specs=pl.BlockSpec((B, _FB), lambda k: (0, k)),
        out_shape=jax.ShapeDtypeStruct((B, FFN), jnp.bfloat16),
        compiler_params=pltpu.CompilerParams(
            dimension_semantics=("parallel",),
            allow_input_fusion=(True, False, False, False),
        ),
    )(g16, W1, W3, rw_col)
    return pl.pallas_call(
        _down_body,
        grid=(_NHB,),
        in_specs=[
            pl.BlockSpec((B, FFN), lambda h: (0, 0)),
            pl.BlockSpec((_HB, FFN), lambda h: (h, 0)),
        ],
        out_specs=pl.BlockSpec((B, _HB), lambda h: (0, h)),
        out_shape=jax.ShapeDtypeStruct((B, HIDDEN), jnp.float32),
        compiler_params=pltpu.CompilerParams(
            dimension_semantics=("parallel",),
            internal_scratch_in_bytes=262144,
        ),
    )(cur16, W2)


def kernel(hidden_states, input_idx, routing_weights, W1, W2, W3):
    idx = input_idx.astype(jnp.int32)
    gathered = _sc_gather(hidden_states, idx)
    g16 = gathered.astype(jnp.bfloat16)
    rw_col = routing_weights.reshape(B, 1)
    return _ffn(g16, rw_col, W1, W2, W3)


# speedup vs baseline: 1.1255x; 1.0297x over previous
import functools

import jax
import jax.numpy as jnp
from jax import lax
from jax.experimental import pallas as pl
from jax.experimental.pallas import tpu as pltpu
from jax.experimental.pallas import tpu_sc as plsc

T = 8192
B = 2048
HIDDEN = 2048
FFN = 7168

_NC = 2
_NS = 16
_NW = _NC * _NS
_BPW = B // _NW
_CHUNK = 16
_NCHUNK = _BPW // _CHUNK


def _sc_gather(table, idx):
    mesh = plsc.VectorSubcoreMesh(core_axis_name="c", subcore_axis_name="s")

    @functools.partial(
        pl.kernel,
        out_type=jax.ShapeDtypeStruct((B, HIDDEN), jnp.float32),
        mesh=mesh,
        scratch_types=[
            pltpu.VMEM((_BPW,), jnp.int32),
            pltpu.VMEM((_CHUNK, HIDDEN), jnp.float32),
            pltpu.VMEM((_CHUNK, HIDDEN), jnp.float32),
            pltpu.SemaphoreType.DMA,
            pltpu.SemaphoreType.DMA,
        ],
    )
    def gather_kernel(table_hbm, idx_hbm, out_hbm, idx_v, rows0, rows1, s0, s1):
        wid = lax.axis_index("s") * _NC + lax.axis_index("c")
        base = wid * _BPW
        pltpu.sync_copy(idx_hbm.at[pl.ds(base, _BPW)], idx_v)
        bufs = (rows0, rows1)
        sems = (s0, s1)
        pltpu.async_copy(table_hbm.at[idx_v.at[pl.ds(0, _CHUNK)]], bufs[0], sems[0])
        for c in range(_NCHUNK):
            if c + 1 < _NCHUNK:
                pltpu.async_copy(
                    table_hbm.at[idx_v.at[pl.ds((c + 1) * _CHUNK, _CHUNK)]],
                    bufs[(c + 1) % 2],
                    sems[(c + 1) % 2],
                )
            pltpu.make_async_copy(
                table_hbm.at[idx_v.at[pl.ds(c * _CHUNK, _CHUNK)]],
                bufs[c % 2],
                sems[c % 2],
            ).wait()
            pltpu.sync_copy(bufs[c % 2], out_hbm.at[pl.ds(base + c * _CHUNK, _CHUNK)])

    return gather_kernel(table, idx)


_NT = (((1,), (1,)), ((), ()))

_FB = 512
_NFB = FFN // _FB
_MB = 512
_NMB = B // _MB


def _gate_body(g_ref, w1_ref, w3_ref, rw_ref, cur_ref):
    w1 = w1_ref[...].astype(jnp.bfloat16)
    w3 = w3_ref[...].astype(jnp.bfloat16)
    _B2 = B // 2
    for i in range(2):
        sl = pl.ds(i * _B2, _B2)
        g = g_ref[sl, :]
        h1 = lax.dot_general(g, w1, _NT, preferred_element_type=jnp.float32)
        h3 = lax.dot_general(g, w3, _NT, preferred_element_type=jnp.float32)
        cur = h1 * jax.nn.sigmoid(h1) * h3 * rw_ref[sl, :]
        cur_ref[sl, :] = cur.astype(jnp.bfloat16)


_HB = 256
_NHB = HIDDEN // _HB


def _down_body(cur_ref, w2_ref, out_ref):
    w2 = w2_ref[...].astype(jnp.bfloat16)
    _B2 = B // 2
    out_ref[:_B2, :] = lax.dot_general(
        cur_ref[:_B2, :], w2, _NT, preferred_element_type=jnp.float32
    )
    out_ref[_B2:, :] = lax.dot_general(
        cur_ref[_B2:, :], w2, _NT, preferred_element_type=jnp.float32
    )


def _ffn(g16, rw_col, W1, W2, W3):
    cur16 = pl.pallas_call(
        _gate_body,
        grid=(_NFB,),
        in_specs=[
            pl.BlockSpec((B, HIDDEN), lambda k: (0, 0)),
            pl.BlockSpec((_FB, HIDDEN), lambda k: (k, 0)),
            pl.BlockSpec((_FB, HIDDEN), lambda k: (k, 0)),
            pl.BlockSpec((B, 1), lambda k: (0, 0)),
        ],
        out_specs=pl.BlockSpec((B, _FB), lambda k: (0, k)),
        out_shape=jax.ShapeDtypeStruct((B, FFN), jnp.bfloat16),
        compiler_params=pltpu.CompilerParams(
            dimension_semantics=("parallel",),
            allow_input_fusion=(True, False, False, False),
        ),
    )(g16, W1, W3, rw_col)
    return pl.pallas_call(
        _down_body,
        grid=(_NHB,),
        in_specs=[
            pl.BlockSpec((B, FFN), lambda h: (0, 0)),
            pl.BlockSpec((_HB, FFN), lambda h: (h, 0)),
        ],
        out_specs=pl.BlockSpec((B, _HB), lambda h: (0, h)),
        out_shape=jax.ShapeDtypeStruct((B, HIDDEN), jnp.float32),
        compiler_params=pltpu.CompilerParams(
            dimension_semantics=("parallel",),
            internal_scratch_in_bytes=262144,
        ),
    )(cur16, W2)


def kernel(hidden_states, input_idx, routing_weights, W1, W2, W3):
    idx = input_idx.astype(jnp.int32)
    gathered = _sc_gather(hidden_states, idx)
    g16 = gathered.astype(jnp.bfloat16)
    rw_col = routing_weights.reshape(B, 1)
    return _ffn(g16, rw_col, W1, W2, W3)
